# Initial kernel scaffold; baseline (speedup 1.0000x reference)
import jax
import jax.numpy as jnp
from jax.experimental import pallas as pl


def kernel(coor, f, coor_q, f_q, W1, g1, b1, W2, g2, b2):
    raise NotImplementedError("write your pallas kernel here")


# speedup vs baseline: 2.5604x; 2.5604x over previous
import jax
import jax.numpy as jnp
from jax.experimental import pallas as pl

_K = 16
_B = 32
_G = 4096
_NQ = 16
_CIN = 384
_P = _K * _NQ


def _topk_sel(s, ncand):
    iota = jax.lax.broadcasted_iota(jnp.int32, (_NQ, ncand), 1)
    rows = []
    for _ in range(_K):
        m = jnp.min(s, axis=1, keepdims=True)
        am = jnp.min(jnp.where(s == m, iota, ncand), axis=1, keepdims=True)
        hit = iota == am
        rows.append(jnp.where(hit, 1.0, 0.0))
        s = jnp.where(hit, jnp.float32(jnp.inf), s)
    return jnp.concatenate(rows, axis=0)


def _blockmax(x):
    z = jnp.maximum(x[:, :-8], x[:, 8:])
    z = jnp.maximum(z[:, :-4], z[:, 4:])
    z = jnp.maximum(z[:, :-2], z[:, 2:])
    z = jnp.maximum(z[:, :-1], z[:, 1:])
    r = jax.lax.broadcasted_iota(jnp.int32, (241, _NQ), 0)
    c = jax.lax.broadcasted_iota(jnp.int32, (241, _NQ), 1)
    epick = jnp.where(r == c * _K, 1.0, 0.0)
    return jnp.dot(z, epick, preferred_element_type=jnp.float32)


def _gn_lrelu(x, gamma, beta, gsize):
    inv_n = 1.0 / (gsize * 256.0)
    parts = []
    for g in range(x.shape[0] // gsize):
        xg = x[g * gsize:(g + 1) * gsize, :]
        mu = jnp.sum(xg) * inv_n
        var = jnp.sum(xg * xg) * inv_n - mu * mu
        y = (xg - mu) * jax.lax.rsqrt(var + 1e-5)
        y = y * gamma[g * gsize:(g + 1) * gsize, :] + beta[g * gsize:(g + 1) * gsize, :]
        parts.append(jnp.where(y >= 0, y, 0.2 * y))
    return jnp.concatenate(parts, axis=0)


_NT = (((1,), (1,)), ((), ()))


def _dgcnn_body(cq_ref, cqt_ref, cgt_ref, f_ref, fq_ref, w1_ref, g1_ref,
                b1_ref, w2_ref, g2_ref, b2_ref, out_ref):
    cq = cq_ref[0]
    cqt = cqt_ref[0]
    cgt = cgt_ref[0]

    pi = jax.lax.broadcasted_iota(jnp.int32, (_P, _NQ), 0)
    qi = jax.lax.broadcasted_iota(jnp.int32, (_P, _NQ), 1)
    eblk = jnp.where(pi // _K == qi, 1.0, 0.0)

    cqb = cq.astype(jnp.bfloat16)
    mm = jnp.dot(cqb, cgt.astype(jnp.bfloat16),
                 preferred_element_type=jnp.float32)
    cqsq = jnp.sum(cq * cq, axis=1, keepdims=True)
    cgsq = jnp.sum(cgt * cgt, axis=0, keepdims=True)
    s1 = (-2.0 * mm + cqsq) + cgsq
    sel = _topk_sel(s1, _G)

    f0 = f_ref[0]
    fq = fq_ref[0]
    fgt = jax.lax.dot_general(sel, f0, _NT,
                              preferred_element_type=jnp.float32)
    fqrep = jax.lax.dot_general(eblk, fq, _NT,
                                preferred_element_type=jnp.float32)
    f1t = jnp.concatenate([fgt - fqrep, fqrep], axis=1).astype(jnp.bfloat16)
    h1 = jax.lax.dot_general(w1_ref[...].astype(jnp.bfloat16), f1t, _NT,
                             preferred_element_type=jnp.float32)
    hfull = _gn_lrelu(h1, g1_ref[...], b1_ref[...], 128)
    h = _blockmax(hfull)

    gram = jnp.dot(cqb, cqt.astype(jnp.bfloat16),
                   preferred_element_type=jnp.float32)
    nt_row = jnp.sum(cqt * cqt, axis=0, keepdims=True)
    s2 = (-2.0 * gram + cqsq) + nt_row
    sel2 = _topk_sel(s2, _NQ)

    hsel = jax.lax.dot_general(sel2, h, _NT,
                               preferred_element_type=jnp.float32)
    hrep = jax.lax.dot_general(eblk, h, _NT,
                               preferred_element_type=jnp.float32)
    f2t = jnp.concatenate([hsel - hrep, hrep], axis=1).astype(jnp.bfloat16)
    h2 = jax.lax.dot_general(w2_ref[...].astype(jnp.bfloat16), f2t, _NT,
                             preferred_element_type=jnp.float32)
    ofull = _gn_lrelu(h2, g2_ref[...], b2_ref[...], 96)
    out_ref[0] = _blockmax(ofull)


@jax.jit
def kernel(coor, f, coor_q, f_q, W1, g1, b1, W2, g2, b2):
    cgt = jnp.pad(jnp.transpose(coor, (0, 2, 1)), ((0, 0), (0, 5), (0, 0)))
    cq = jnp.pad(coor_q, ((0, 0), (0, 0), (0, 5)))
    cqt = jnp.pad(jnp.transpose(coor_q, (0, 2, 1)), ((0, 0), (0, 5), (0, 0)))
    g1c = g1.reshape(512, 1)
    b1c = b1.reshape(512, 1)
    g2c = g2.reshape(384, 1)
    b2c = b2.reshape(384, 1)

    bcast = lambda *shape: pl.BlockSpec(shape, lambda b: (0,) * len(shape))
    perb = lambda *shape: pl.BlockSpec(shape, lambda b: (b,) + (0,) * (len(shape) - 1))

    return pl.pallas_call(
        _dgcnn_body,
        grid=(_B,),
        in_specs=[
            perb(1, _NQ, 8),
            perb(1, 8, _NQ),
            perb(1, 8, _G),
            perb(1, _CIN, _G),
            perb(1, _CIN, _NQ),
            bcast(512, 768),
            bcast(512, 1),
            bcast(512, 1),
            bcast(384, 1024),
            bcast(384, 1),
            bcast(384, 1),
        ],
        out_specs=perb(1, 384, _NQ),
        out_shape=jax.ShapeDtypeStruct((_B, 384, _NQ), jnp.float32),
    )(cq, cqt, cgt, f, f_q, W1, g1c, b1c, W2, g2c, b2c)


# speedup vs baseline: 3.3655x; 1.3145x over previous
import jax
import jax.numpy as jnp
from jax.experimental import pallas as pl
from jax.experimental.pallas import tpu as pltpu

_K = 16
_B = 32
_G = 4096
_NQ = 16
_CIN = 384
_P = _K * _NQ
_NT = (((1,), (1,)), ((), ()))


def _rowmin(x):
    n = x.shape[1]
    while n > 128:
        n //= 2
        x = jnp.minimum(x[:, :n], x[:, n:])
    return jnp.min(x, axis=1, keepdims=True)


def _topk_sel(s, ncand):
    iota = jax.lax.broadcasted_iota(jnp.int32, (_NQ, ncand), 1)
    rows = []
    for _ in range(_K):
        m = _rowmin(s)
        am = _rowmin(jnp.where(s == m, iota, ncand))
        hit = iota == am
        rows.append(jnp.where(hit, 1.0, 0.0))
        s = jnp.where(hit, jnp.float32(jnp.inf), s)
    return jnp.concatenate(rows, axis=0)


def _blockmax(x):
    z = jnp.maximum(x[:, :-8], x[:, 8:])
    z = jnp.maximum(z[:, :-4], z[:, 4:])
    z = jnp.maximum(z[:, :-2], z[:, 2:])
    z = jnp.maximum(z[:, :-1], z[:, 1:])
    r = jax.lax.broadcasted_iota(jnp.int32, (241, _NQ), 0)
    c = jax.lax.broadcasted_iota(jnp.int32, (241, _NQ), 1)
    epick = jnp.where(r == c * _K, 1.0, 0.0)
    return jnp.dot(z, epick, preferred_element_type=jnp.float32)


def _gn_lrelu(x, gamma, beta, gsize):
    inv_n = 1.0 / (gsize * 256.0)
    parts = []
    for g in range(x.shape[0] // gsize):
        xg = x[g * gsize:(g + 1) * gsize, :]
        mu = jnp.sum(xg) * inv_n
        var = jnp.sum(xg * xg) * inv_n - mu * mu
        y = (xg - mu) * jax.lax.rsqrt(var + 1e-5)
        y = y * gamma[g * gsize:(g + 1) * gsize, :] + beta[g * gsize:(g + 1) * gsize, :]
        parts.append(jnp.where(y >= 0, y, 0.2 * y))
    return jnp.concatenate(parts, axis=0)


def _dgcnn_body(cq_ref, cqt_ref, cgt_ref, f_ref, fq_ref, w1_ref, g1_ref,
                b1_ref, w2_ref, g2_ref, b2_ref, out_ref, sel_ref, sel2_ref):
    b = pl.program_id(0)
    wslot = jax.lax.rem(b, 2)
    rslot = 1 - wslot

    pi = jax.lax.broadcasted_iota(jnp.int32, (_P, _NQ), 0)
    qi = jax.lax.broadcasted_iota(jnp.int32, (_P, _NQ), 1)
    eblk = jnp.where(pi // _K == qi, 1.0, 0.0)

    sel = sel_ref[rslot]
    sel2 = sel2_ref[rslot]
    f0 = f_ref[0]
    fq = fq_ref[0]
    fgt = jax.lax.dot_general(sel, f0, _NT,
                              preferred_element_type=jnp.float32)
    fqrep = jax.lax.dot_general(eblk, fq, _NT,
                                preferred_element_type=jnp.float32)
    f1t = jnp.concatenate([fgt - fqrep, fqrep], axis=1).astype(jnp.bfloat16)
    h1 = jax.lax.dot_general(w1_ref[...].astype(jnp.bfloat16), f1t, _NT,
                             preferred_element_type=jnp.float32)
    hfull = _gn_lrelu(h1, g1_ref[...], b1_ref[...], 128)
    h = _blockmax(hfull)

    hsel = jax.lax.dot_general(sel2, h, _NT,
                               preferred_element_type=jnp.float32)
    hrep = jax.lax.dot_general(eblk, h, _NT,
                               preferred_element_type=jnp.float32)
    f2t = jnp.concatenate([hsel - hrep, hrep], axis=1).astype(jnp.bfloat16)
    h2 = jax.lax.dot_general(w2_ref[...].astype(jnp.bfloat16), f2t, _NT,
                             preferred_element_type=jnp.float32)
    ofull = _gn_lrelu(h2, g2_ref[...], b2_ref[...], 96)
    out_ref[0] = _blockmax(ofull)

    cq = cq_ref[0]
    cqt = cqt_ref[0]
    cgt = cgt_ref[0]

    cqb = cq.astype(jnp.bfloat16)
    mm = jnp.dot(cqb, cgt.astype(jnp.bfloat16),
                 preferred_element_type=jnp.float32)
    cqsq = jnp.sum(cq * cq, axis=1, keepdims=True)
    cgsq = jnp.sum(cgt * cgt, axis=0, keepdims=True)
    s1 = (-2.0 * mm + cqsq) + cgsq
    sel_ref[wslot] = _topk_sel(s1, _G)

    gram = jnp.dot(cqb, cqt.astype(jnp.bfloat16),
                   preferred_element_type=jnp.float32)
    nt_row = jnp.sum(cqt * cqt, axis=0, keepdims=True)
    s2 = (-2.0 * gram + cqsq) + nt_row
    sel2_ref[wslot] = _topk_sel(s2, _NQ)


@jax.jit
def kernel(coor, f, coor_q, f_q, W1, g1, b1, W2, g2, b2):
    cgt = jnp.pad(jnp.transpose(coor, (0, 2, 1)), ((0, 0), (0, 5), (0, 0)))
    cq = jnp.pad(coor_q, ((0, 0), (0, 0), (0, 5)))
    cqt = jnp.pad(jnp.transpose(coor_q, (0, 2, 1)), ((0, 0), (0, 5), (0, 0)))
    g1c = g1.reshape(512, 1)
    b1c = b1.reshape(512, 1)
    g2c = g2.reshape(384, 1)
    b2c = b2.reshape(384, 1)

    def topk_idx(*shape):
        return pl.BlockSpec(
            shape, lambda b: (jnp.minimum(b, _B - 1),) + (0,) * (len(shape) - 1))

    def dense_idx(*shape):
        return pl.BlockSpec(
            shape, lambda b: (jnp.maximum(b - 1, 0),) + (0,) * (len(shape) - 1))

    bcast = lambda *shape: pl.BlockSpec(shape, lambda b: (0,) * len(shape))

    return pl.pallas_call(
        _dgcnn_body,
        grid=(_B + 1,),
        in_specs=[
            topk_idx(1, _NQ, 8),
            topk_idx(1, 8, _NQ),
            topk_idx(1, 8, _G),
            dense_idx(1, _CIN, _G),
            dense_idx(1, _CIN, _NQ),
            bcast(512, 768),
            bcast(512, 1),
            bcast(512, 1),
            bcast(384, 1024),
            bcast(384, 1),
            bcast(384, 1),
        ],
        out_specs=dense_idx(1, 384, _NQ),
        out_shape=jax.ShapeDtypeStruct((_B, 384, _NQ), jnp.float32),
        scratch_shapes=[
            pltpu.VMEM((2, _P, _G), jnp.float32),
            pltpu.VMEM((2, _P, _NQ), jnp.float32),
        ],
    )(cq, cqt, cgt, f, f_q, W1, g1c, b1c, W2, g2c, b2c)


# speedup vs baseline: 4.2899x; 1.2747x over previous
import jax
import jax.numpy as jnp
from jax.experimental import pallas as pl
from jax.experimental.pallas import tpu as pltpu

_K = 16
_B = 32
_G = 4096
_NQ = 16
_CIN = 384
_P = _K * _NQ
_NT = (((1,), (1,)), ((), ()))


def _rowmin(x):
    n = x.shape[1]
    while n > 128:
        n //= 2
        x = jnp.minimum(x[:, :n], x[:, n:])
    return jnp.min(x, axis=1, keepdims=True)


def _topk_half(s, ncand, ref, slot, row_off):
    iota = jax.lax.broadcasted_iota(jnp.int32, (8, ncand), 1)
    for r in range(_K):
        m = _rowmin(s)
        am = _rowmin(jnp.where(s == m, iota, ncand))
        hit = iota == am
        ref[slot, r * _K + row_off:r * _K + row_off + 8, :] = (
            jnp.where(hit, 1.0, 0.0))
        s = jnp.where(hit, jnp.float32(jnp.inf), s)


def _topk_store(s, ncand, ref, slot):
    _topk_half(s[:8, :], ncand, ref, slot, 0)
    _topk_half(s[8:, :], ncand, ref, slot, 8)


def _blockmax(x):
    z = jnp.maximum(x[:, :-8], x[:, 8:])
    z = jnp.maximum(z[:, :-4], z[:, 4:])
    z = jnp.maximum(z[:, :-2], z[:, 2:])
    z = jnp.maximum(z[:, :-1], z[:, 1:])
    r = jax.lax.broadcasted_iota(jnp.int32, (241, _NQ), 0)
    c = jax.lax.broadcasted_iota(jnp.int32, (241, _NQ), 1)
    epick = jnp.where(r == c * _K, 1.0, 0.0)
    return jnp.dot(z, epick, preferred_element_type=jnp.float32)


def _gn_lrelu(x, gamma, beta, gsize):
    inv_n = 1.0 / (gsize * 256.0)
    parts = []
    for g in range(x.shape[0] // gsize):
        xg = x[g * gsize:(g + 1) * gsize, :]
        mu = jnp.sum(xg) * inv_n
        var = jnp.sum(xg * xg) * inv_n - mu * mu
        y = (xg - mu) * jax.lax.rsqrt(var + 1e-5)
        y = y * gamma[g * gsize:(g + 1) * gsize, :] + beta[g * gsize:(g + 1) * gsize, :]
        parts.append(jnp.where(y >= 0, y, 0.2 * y))
    return jnp.concatenate(parts, axis=0)


def _dgcnn_body(cq_ref, cqt_ref, cgt_ref, f_ref, fq_ref, w1_ref, g1_ref,
                b1_ref, w2_ref, g2_ref, b2_ref, out_ref, sel_ref, sel2_ref):
    b = pl.program_id(0)
    wslot = jax.lax.rem(b, 2)
    rslot = 1 - wslot


    cq = cq_ref[0]
    cqt = cqt_ref[0]
    cgt = cgt_ref[0]
    cqb = cq.astype(jnp.bfloat16)
    mm = jnp.dot(cqb, cgt.astype(jnp.bfloat16),
                 preferred_element_type=jnp.float32)
    gram = jnp.dot(cqb, cqt.astype(jnp.bfloat16),
                   preferred_element_type=jnp.float32)
    cqsq = jnp.sum(cq * cq, axis=1, keepdims=True)
    cgsq = jnp.sum(cgt * cgt, axis=0, keepdims=True)
    s1 = (-2.0 * mm + cqsq) + cgsq
    nt_row = jnp.sum(cqt * cqt, axis=0, keepdims=True)
    s2 = (-2.0 * gram + cqsq) + nt_row

    pi = jax.lax.broadcasted_iota(jnp.int32, (_P, _NQ), 0)
    qi = jax.lax.broadcasted_iota(jnp.int32, (_P, _NQ), 1)
    eblk = jnp.where(pi // _K == qi, 1.0, 0.0)

    sel = sel_ref[rslot]
    sel2 = sel2_ref[rslot]
    f0 = f_ref[0]
    fq = fq_ref[0]
    fgt = jax.lax.dot_general(sel, f0, _NT,
                              preferred_element_type=jnp.float32)
    fqrep = jax.lax.dot_general(eblk, fq, _NT,
                                preferred_element_type=jnp.float32)
    f1t = jnp.concatenate([fgt - fqrep, fqrep], axis=1).astype(jnp.bfloat16)
    h1 = jax.lax.dot_general(w1_ref[...], f1t, _NT,
                             preferred_element_type=jnp.float32)

    _topk_store(s1, _G, sel_ref, wslot)

    hfull = _gn_lrelu(h1, g1_ref[...], b1_ref[...], 128)
    h = _blockmax(hfull)
    hsel = jax.lax.dot_general(sel2, h, _NT,
                               preferred_element_type=jnp.float32)
    hrep = jax.lax.dot_general(eblk, h, _NT,
                               preferred_element_type=jnp.float32)
    f2t = jnp.concatenate([hsel - hrep, hrep], axis=1).astype(jnp.bfloat16)
    h2 = jax.lax.dot_general(w2_ref[...], f2t, _NT,
                             preferred_element_type=jnp.float32)

    _topk_store(s2, _NQ, sel2_ref, wslot)

    ofull = _gn_lrelu(h2, g2_ref[...], b2_ref[...], 96)
    out_ref[0] = _blockmax(ofull)


@jax.jit
def kernel(coor, f, coor_q, f_q, W1, g1, b1, W2, g2, b2):
    cgt = jnp.pad(jnp.transpose(coor, (0, 2, 1)), ((0, 0), (0, 5), (0, 0)))
    cq = jnp.pad(coor_q, ((0, 0), (0, 0), (0, 5)))
    cqt = jnp.pad(jnp.transpose(coor_q, (0, 2, 1)), ((0, 0), (0, 5), (0, 0)))
    w1b16 = W1.astype(jnp.bfloat16)
    w2b16 = W2.astype(jnp.bfloat16)
    g1c = g1.reshape(512, 1)
    b1c = b1.reshape(512, 1)
    g2c = g2.reshape(384, 1)
    b2c = b2.reshape(384, 1)

    def topk_idx(*shape):
        return pl.BlockSpec(
            shape, lambda b: (jnp.minimum(b, _B - 1),) + (0,) * (len(shape) - 1))

    def dense_idx(*shape):
        return pl.BlockSpec(
            shape, lambda b: (jnp.maximum(b - 1, 0),) + (0,) * (len(shape) - 1))

    bcast = lambda *shape: pl.BlockSpec(shape, lambda b: (0,) * len(shape))

    return pl.pallas_call(
        _dgcnn_body,
        grid=(_B + 1,),
        in_specs=[
            topk_idx(1, _NQ, 8),
            topk_idx(1, 8, _NQ),
            topk_idx(1, 8, _G),
            dense_idx(1, _CIN, _G),
            dense_idx(1, _CIN, _NQ),
            bcast(512, 768),
            bcast(512, 1),
            bcast(512, 1),
            bcast(384, 1024),
            bcast(384, 1),
            bcast(384, 1),
        ],
        out_specs=dense_idx(1, 384, _NQ),
        out_shape=jax.ShapeDtypeStruct((_B, 384, _NQ), jnp.float32),
        scratch_shapes=[
            pltpu.VMEM((2, _P, _G), jnp.float32),
            pltpu.VMEM((2, _P, _NQ), jnp.float32),
        ],
    )(cq, cqt, cgt, f, f_q, w1b16, g1c, b1c, w2b16, g2c, b2c)
